# baseline (speedup 1.0000x reference)
import functools

import jax
import jax.numpy as jnp
from jax import lax
from jax.experimental import pallas as pl
from jax.experimental.pallas import tpu as pltpu
from jax.experimental.pallas import tpu_sc as plsc

N = 10000
E = 320000
F = 128
KHOPS = 3
G = 100
GP = 112
GACC = 128
NC = 2
NS = 16
NW = NC * NS
CH = 128
NCHUNK = E // CH
RA = 624
RL = N - (NS - 1) * RA
BP = 10240
NBCH = BP // CH
BLK = 1024
BPW = 10240
E2 = NW * BPW
ROWBLK = 1000
NBLK = N // ROWBLK

_MESH = plsc.VectorSubcoreMesh(core_axis_name="c", subcore_axis_name="s",
                               num_cores=NC, num_subcores=NS)


def _rows_copy(src, dst, sid):
    off = pl.multiple_of(sid * RA, 8)

    @pl.when(sid < NS - 1)
    def _():
        pltpu.sync_copy(src.at[pl.ds(off, RA)], dst.at[pl.ds(off, RA)])

    @pl.when(sid == NS - 1)
    def _():
        pltpu.sync_copy(src.at[pl.ds((NS - 1) * RA, RL)],
                        dst.at[pl.ds((NS - 1) * RA, RL)])


def _chunks_for(w, total):
    full = total // NW
    extra = total - full * NW
    return jnp.where(w < extra, full + 1, full)


@functools.partial(
    pl.kernel,
    out_type=(
        jax.ShapeDtypeStruct((NC, N, F), jnp.float32),
        jax.ShapeDtypeStruct((NC, GACC, F), jnp.float32),
    ),
    mesh=_MESH,
    scratch_types=[
        pltpu.VMEM_SHARED((N, F), jnp.float32),
        pltpu.VMEM_SHARED((GACC, F), jnp.float32),
        pltpu.VMEM((1, CH), jnp.int32),
        pltpu.VMEM((CH, F), jnp.float32),
    ],
)
def _deg_kernel(dst_hbm, bids_hbm, ones_hbm, zeros_hbm,
                deg_out, cnt_out, acc_deg, acc_cnt, idx_v, ones_v):
    cid = lax.axis_index("c")
    sid = lax.axis_index("s")
    w = cid * NS + sid

    _rows_copy(zeros_hbm, acc_deg, sid)

    @pl.when(sid == 0)
    def _():
        pltpu.sync_copy(zeros_hbm.at[pl.ds(0, GACC)], acc_cnt)

    pltpu.sync_copy(ones_hbm, ones_v)
    plsc.subcore_barrier()

    def edge_body(j, _):
        ci = w + j * NW
        off = pl.multiple_of(ci * CH, CH)
        pltpu.sync_copy(dst_hbm.at[pl.ds(off, CH)], idx_v.at[0])
        pltpu.sync_copy(ones_v, acc_deg.at[idx_v.at[0]], add=True)
        return _

    lax.fori_loop(0, _chunks_for(w, NCHUNK), edge_body, 0)

    def bid_body(j, _):
        ci = w + j * NW
        off = pl.multiple_of(ci * CH, CH)
        pltpu.sync_copy(bids_hbm.at[pl.ds(off, CH)], idx_v.at[0])
        pltpu.sync_copy(ones_v, acc_cnt.at[idx_v.at[0]], add=True)
        return _

    lax.fori_loop(0, _chunks_for(w, NBCH), bid_body, 0)
    plsc.subcore_barrier()

    _rows_copy(acc_deg, deg_out.at[cid], sid)

    @pl.when(sid == 0)
    def _():
        pltpu.sync_copy(acc_cnt, cnt_out.at[cid])


@functools.partial(
    pl.kernel,
    out_type=jax.ShapeDtypeStruct((NC, N, F), jnp.float32),
    mesh=_MESH,
    scratch_types=[
        pltpu.VMEM_SHARED((N + CH, F), jnp.float32),
        pltpu.VMEM((CH,), jnp.int32),
        pltpu.VMEM((BLK // CH, CH), jnp.int32),
        pltpu.VMEM((CH, F), jnp.float32),
        pltpu.VMEM((CH, F), jnp.float32),
        pltpu.SemaphoreType.DMA,
        pltpu.SemaphoreType.DMA,
    ],
)
def _hop_kernel(g_hbm, src_hbm, dst_hbm, zeros_hbm,
                p_out, acc, sidx_v, didx_v, rows0, rows1, sem0, sem1):
    cid = lax.axis_index("c")
    sid = lax.axis_index("s")
    w = cid * NS + sid

    @pl.when(cid == 0)
    def _():
        _rows_copy(g_hbm, acc, sid)

    @pl.when(cid != 0)
    def _():
        _rows_copy(zeros_hbm, acc, sid)

    plsc.subcore_barrier()

    bufs = (rows0, rows1)
    sems = (sem0, sem1)
    nch = BLK // CH

    def block_body(b, carry):
        brow = pl.multiple_of(w * (BPW // CH) + b * nch, 8)
        pltpu.sync_copy(dst_hbm.at[pl.ds(brow, nch)], didx_v)
        for c in range(nch):
            pltpu.sync_copy(src_hbm.at[brow + c], sidx_v)
            pltpu.async_copy(g_hbm.at[sidx_v], bufs[0], sems[0]).wait()
            pltpu.sync_copy(bufs[0], acc.at[didx_v.at[c]], add=True)
        return carry

    lax.fori_loop(0, BPW // BLK, block_body, 0)
    plsc.subcore_barrier()

    _rows_copy(acc, p_out.at[cid], sid)


@functools.partial(
    pl.kernel,
    out_type=(
        jax.ShapeDtypeStruct((GP, F), jnp.float32),
        jax.ShapeDtypeStruct((GP, F), jnp.float32),
    ),
    mesh=_MESH,
    scratch_types=[
        pltpu.VMEM((NC * GACC * F,), jnp.float32),
        pltpu.VMEM((GP * 2,), jnp.int32),
        pltpu.VMEM((GP,), jnp.int32),
        pltpu.VMEM((GP,), jnp.int32),
        pltpu.VMEM((GP, F), jnp.float32),
        pltpu.VMEM((GP, F), jnp.float32),
        pltpu.SemaphoreType.DMA,
    ],
    compiler_params=pltpu.CompilerParams(needs_layout_passes=False),
)
def _head_kernel(h_hbm, cnt_hbm, set_hbm,
                 xsa_out, xsb_out, cnt_v, set_v, siba_v, sibb_v, ra_v, rb_v, sem):
    cid = lax.axis_index("c")
    sid = lax.axis_index("s")

    @pl.when(jnp.logical_and(cid == 0, sid == 0))
    def _():
        pltpu.sync_copy(cnt_hbm, cnt_v)
        pltpu.sync_copy(set_hbm, set_v)
        lane = lax.iota(jnp.int32, 16)
        carry = jnp.zeros((), jnp.int32)
        for c in range(GP // 16):
            rows = lane + c * 16
            cnt = (plsc.load_gather(cnt_v, [rows * F])
                   + plsc.load_gather(cnt_v, [rows * F + GACC * F]))
            cnt_i = cnt.astype(jnp.int32)
            bases = plsc.cumsum(cnt_i) - cnt_i + carry
            carry = carry + jnp.sum(cnt_i)
            sa = plsc.load_gather(set_v, [rows * 2])
            sb = plsc.load_gather(set_v, [rows * 2 + 1])
            siba_v[pl.ds(c * 16, 16)] = jnp.minimum(bases + sa, N - 1)
            sibb_v[pl.ds(c * 16, 16)] = jnp.minimum(bases + sb, N - 1)
        pltpu.async_copy(h_hbm.at[siba_v], ra_v, sem).wait()
        pltpu.async_copy(h_hbm.at[sibb_v], rb_v, sem).wait()
        pltpu.sync_copy(ra_v, xsa_out)
        pltpu.sync_copy(rb_v, xsb_out)


def _init_body(degp_ref, x_ref, sc_ref, g0_ref):
    d = degp_ref[0, :, 0:1] + degp_ref[1, :, 0:1] + 1.0
    dis = lax.rsqrt(d)
    v = 1.0 / d
    r = d * dis
    sc_ref[:, 0:1] = v
    sc_ref[:, 1:2] = dis
    sc_ref[:, 2:3] = r
    sc_ref[:, 3:8] = jnp.zeros_like(sc_ref[:, 3:8])
    g0_ref[...] = dis * x_ref[...]


def _tc_init(degp, x):
    return pl.pallas_call(
        _init_body,
        grid=(NBLK,),
        in_specs=[
            pl.BlockSpec((NC, ROWBLK, F), lambda i: (0, i, 0)),
            pl.BlockSpec((ROWBLK, F), lambda i: (i, 0)),
        ],
        out_specs=[
            pl.BlockSpec((ROWBLK, 8), lambda i: (i, 0)),
            pl.BlockSpec((ROWBLK, F), lambda i: (i, 0)),
        ],
        out_shape=[
            jax.ShapeDtypeStruct((N, 8), jnp.float32),
            jax.ShapeDtypeStruct((N, F), jnp.float32),
        ],
    )(degp, x)


def _hopmid_body(p_ref, sc_ref, g_ref):
    v = sc_ref[:, 0:1]
    g_ref[...] = v * (p_ref[0] + p_ref[1])


def _tc_hopmid(p, sc):
    return pl.pallas_call(
        _hopmid_body,
        grid=(NBLK,),
        in_specs=[
            pl.BlockSpec((NC, ROWBLK, F), lambda i: (0, i, 0)),
            pl.BlockSpec((ROWBLK, 8), lambda i: (i, 0)),
        ],
        out_specs=pl.BlockSpec((ROWBLK, F), lambda i: (i, 0)),
        out_shape=jax.ShapeDtypeStruct((N, F), jnp.float32),
    )(p, sc)


def _layer_body(g0_ref, g1_ref, g2_ref, g3_ref, sc_ref, w_ref, b_ref,
                lng_ref, lnb_ref, h_ref, gn_ref):
    acc = jnp.dot(g0_ref[...], w_ref[0:F, :], preferred_element_type=jnp.float32)
    acc += jnp.dot(g1_ref[...], w_ref[F:2 * F, :], preferred_element_type=jnp.float32)
    acc += jnp.dot(g2_ref[...], w_ref[2 * F:3 * F, :], preferred_element_type=jnp.float32)
    acc += jnp.dot(g3_ref[...], w_ref[3 * F:4 * F, :], preferred_element_type=jnp.float32)
    y = sc_ref[:, 2:3] * acc + b_ref[...]
    y = jnp.maximum(y, 0.0)
    mu = jnp.mean(y, axis=-1, keepdims=True)
    yc = y - mu
    var = jnp.mean(yc * yc, axis=-1, keepdims=True)
    h = lng_ref[...] * yc * lax.rsqrt(var + 1e-5) + lnb_ref[...]
    h_ref[...] = h
    gn_ref[...] = sc_ref[:, 1:2] * h


def _tc_layer(g0, g1, g2, g3, sc, wc, b, lng, lnb):
    return pl.pallas_call(
        _layer_body,
        grid=(NBLK,),
        in_specs=[
            pl.BlockSpec((ROWBLK, F), lambda i: (i, 0)),
            pl.BlockSpec((ROWBLK, F), lambda i: (i, 0)),
            pl.BlockSpec((ROWBLK, F), lambda i: (i, 0)),
            pl.BlockSpec((ROWBLK, F), lambda i: (i, 0)),
            pl.BlockSpec((ROWBLK, 8), lambda i: (i, 0)),
            pl.BlockSpec(((KHOPS + 1) * F, F), lambda i: (0, 0)),
            pl.BlockSpec((1, F), lambda i: (0, 0)),
            pl.BlockSpec((1, F), lambda i: (0, 0)),
            pl.BlockSpec((1, F), lambda i: (0, 0)),
        ],
        out_specs=[
            pl.BlockSpec((ROWBLK, F), lambda i: (i, 0)),
            pl.BlockSpec((ROWBLK, F), lambda i: (i, 0)),
        ],
        out_shape=[
            jax.ShapeDtypeStruct((N, F), jnp.float32),
            jax.ShapeDtypeStruct((N, F), jnp.float32),
        ],
    )(g0, g1, g2, g3, sc, wc, b, lng, lnb)


def _head_body(xsa_ref, xsb_ref, mw_ref, mb_ref, f1w_ref, f1b_ref,
               f2w_ref, f2b_ref, out_ref):
    a = xsa_ref[...]
    b = xsb_ref[...]
    df = jnp.abs(a - b)
    mn = 0.5 * (a + b)
    mx = jnp.maximum(a, b)
    m = jnp.dot(df, mw_ref[0:F, :], preferred_element_type=jnp.float32)
    m += jnp.dot(mn, mw_ref[F:2 * F, :], preferred_element_type=jnp.float32)
    m += jnp.dot(mx, mw_ref[2 * F:3 * F, :], preferred_element_type=jnp.float32)
    m += mb_ref[...]
    f = jnp.maximum(jnp.dot(m, f1w_ref[...], preferred_element_type=jnp.float32)
                    + f1b_ref[...], 0.0)
    o = jnp.dot(f, f2w_ref[...], preferred_element_type=jnp.float32) + f2b_ref[...]
    out_ref[...] = o[0:G, :]


def _tc_head(xsa, xsb, mw, mb, f1w, f1b, f2w, f2b):
    return pl.pallas_call(
        _head_body,
        grid=(1,),
        in_specs=[
            pl.BlockSpec((GP, F), lambda i: (0, 0)),
            pl.BlockSpec((GP, F), lambda i: (0, 0)),
            pl.BlockSpec((3 * F, F), lambda i: (0, 0)),
            pl.BlockSpec((1, F), lambda i: (0, 0)),
            pl.BlockSpec((F, F), lambda i: (0, 0)),
            pl.BlockSpec((1, F), lambda i: (0, 0)),
            pl.BlockSpec((F, F), lambda i: (0, 0)),
            pl.BlockSpec((1, F), lambda i: (0, 0)),
        ],
        out_specs=pl.BlockSpec((G, F), lambda i: (0, 0)),
        out_shape=jax.ShapeDtypeStruct((G, F), jnp.float32),
    )(xsa, xsb, mw, mb, f1w, f1b, f2w, f2b)


def kernel(x, edge_index, set_indices, batch_ids, num_graphs, W0, b0, W1, b1,
           ln0_g, ln0_b, ln1_g, ln1_b, merger_W, merger_b, ff1_W, ff1_b,
           ff2_W, ff2_b):
    src = edge_index[0]
    dst = edge_index[1]
    epad = E2 - E
    src_pad = jnp.concatenate(
        [src, jnp.zeros((epad,), jnp.int32)]).reshape(E2 // CH, CH)
    dst_pad = jnp.concatenate(
        [dst, N + (jnp.arange(epad, dtype=jnp.int32) % CH)]).reshape(E2 // CH, CH)
    bids_pad = jnp.concatenate(
        [batch_ids, jnp.full((BP - N,), GACC - 1, jnp.int32)])
    set_pad = jnp.concatenate(
        [set_indices, jnp.zeros((GP - G, 2), jnp.int32)], axis=0)
    zeros_nf = jnp.zeros((N, F), jnp.float32)
    ones_chf = jnp.ones((CH, F), jnp.float32)

    degp, cntp = _deg_kernel(dst, bids_pad, ones_chf, zeros_nf)
    sc, g0 = _tc_init(degp, x)

    def run_layer(g_first, wc, b, lng, lnb):
        gs = [g_first]
        for _ in range(KHOPS):
            p = _hop_kernel(gs[-1], src_pad, dst_pad, zeros_nf)
            gs.append(_tc_hopmid(p, sc))
        return _tc_layer(gs[0], gs[1], gs[2], gs[3], sc, wc,
                         b.reshape(1, F), lng.reshape(1, F), lnb.reshape(1, F))

    wc0 = W0.reshape((KHOPS + 1) * F, F)
    wc1 = W1.reshape((KHOPS + 1) * F, F)
    h1, g1_first = run_layer(g0, wc0, b0, ln0_g, ln0_b)
    h2, _ = run_layer(g1_first, wc1, b1, ln1_g, ln1_b)

    xsa, xsb = _head_kernel(h2, cntp.reshape(-1), set_pad.reshape(-1))
    out = _tc_head(xsa, xsb, merger_W, merger_b.reshape(1, F),
                   ff1_W, ff1_b.reshape(1, F), ff2_W, ff2_b.reshape(1, F))
    return out


# speedup vs baseline: 2.3535x; 2.3535x over previous
import functools

import jax
import jax.numpy as jnp
from jax import lax
from jax.experimental import pallas as pl
from jax.experimental.pallas import tpu as pltpu
from jax.experimental.pallas import tpu_sc as plsc

N = 10000
E = 320000
F = 128
KHOPS = 3
G = 100
GP = 112
GACC = 128
NC = 2
NS = 16
NW = NC * NS
CH = 128
NCHUNK = E // CH
RA = 624
RL = N - (NS - 1) * RA
BP = 10240
NBCH = BP // CH
BLK = 1024
BPW = 10240
E2 = NW * BPW
ROWBLK = 1000
NBLK = N // ROWBLK

_MESH = plsc.VectorSubcoreMesh(core_axis_name="c", subcore_axis_name="s",
                               num_cores=NC, num_subcores=NS)


def _rows_copy(src, dst, sid):
    off = pl.multiple_of(sid * RA, 8)

    @pl.when(sid < NS - 1)
    def _():
        pltpu.sync_copy(src.at[pl.ds(off, RA)], dst.at[pl.ds(off, RA)])

    @pl.when(sid == NS - 1)
    def _():
        pltpu.sync_copy(src.at[pl.ds((NS - 1) * RA, RL)],
                        dst.at[pl.ds((NS - 1) * RA, RL)])


def _chunks_for(w, total):
    full = total // NW
    extra = total - full * NW
    return jnp.where(w < extra, full + 1, full)


@functools.partial(
    pl.kernel,
    out_type=(
        jax.ShapeDtypeStruct((NC, N, F), jnp.float32),
        jax.ShapeDtypeStruct((NC, GACC, F), jnp.float32),
    ),
    mesh=_MESH,
    scratch_types=[
        pltpu.VMEM_SHARED((N, F), jnp.float32),
        pltpu.VMEM_SHARED((GACC, F), jnp.float32),
        pltpu.VMEM((1, CH), jnp.int32),
        pltpu.VMEM((CH, F), jnp.float32),
    ],
)
def _deg_kernel(dst_hbm, bids_hbm, ones_hbm, zeros_hbm,
                deg_out, cnt_out, acc_deg, acc_cnt, idx_v, ones_v):
    cid = lax.axis_index("c")
    sid = lax.axis_index("s")
    w = cid * NS + sid

    _rows_copy(zeros_hbm, acc_deg, sid)

    @pl.when(sid == 0)
    def _():
        pltpu.sync_copy(zeros_hbm.at[pl.ds(0, GACC)], acc_cnt)

    pltpu.sync_copy(ones_hbm, ones_v)
    plsc.subcore_barrier()

    def edge_body(j, _):
        ci = w + j * NW
        off = pl.multiple_of(ci * CH, CH)
        pltpu.sync_copy(dst_hbm.at[pl.ds(off, CH)], idx_v.at[0])
        pltpu.sync_copy(ones_v, acc_deg.at[idx_v.at[0]], add=True)
        return _

    lax.fori_loop(0, _chunks_for(w, NCHUNK), edge_body, 0)

    def bid_body(j, _):
        ci = w + j * NW
        off = pl.multiple_of(ci * CH, CH)
        pltpu.sync_copy(bids_hbm.at[pl.ds(off, CH)], idx_v.at[0])
        pltpu.sync_copy(ones_v, acc_cnt.at[idx_v.at[0]], add=True)
        return _

    lax.fori_loop(0, _chunks_for(w, NBCH), bid_body, 0)
    plsc.subcore_barrier()

    _rows_copy(acc_deg, deg_out.at[cid], sid)

    @pl.when(sid == 0)
    def _():
        pltpu.sync_copy(acc_cnt, cnt_out.at[cid])


@functools.partial(
    pl.kernel,
    out_type=jax.ShapeDtypeStruct((NC, N, F), jnp.float32),
    mesh=_MESH,
    scratch_types=[
        pltpu.VMEM_SHARED((N + CH, F), jnp.float32),
        pltpu.VMEM((CH,), jnp.int32),
        pltpu.VMEM((BLK // CH, CH), jnp.int32),
        pltpu.VMEM((CH, F), jnp.float32),
        pltpu.VMEM((CH, F), jnp.float32),
        pltpu.SemaphoreType.DMA,
        pltpu.SemaphoreType.DMA,
    ],
)
def _hop_kernel(g_hbm, src_hbm, dst_hbm, zeros_hbm,
                p_out, acc, sidx_v, didx_v, rows0, rows1, sem0, sem1):
    cid = lax.axis_index("c")
    sid = lax.axis_index("s")
    w = cid * NS + sid

    @pl.when(cid == 0)
    def _():
        _rows_copy(g_hbm, acc, sid)

    @pl.when(cid != 0)
    def _():
        _rows_copy(zeros_hbm, acc, sid)

    plsc.subcore_barrier()

    bufs = (rows0, rows1)
    sems = (sem0, sem1)
    nch = BLK // CH

    def block_body(b, carry):
        brow = pl.multiple_of(w * (BPW // CH) + b * nch, 8)
        pltpu.sync_copy(dst_hbm.at[pl.ds(brow, nch)], didx_v)
        for c in range(nch):
            pltpu.sync_copy(src_hbm.at[brow + c], sidx_v)
            pltpu.async_copy(g_hbm.at[sidx_v], bufs[0], sems[0]).wait()
            pltpu.sync_copy(bufs[0], acc.at[didx_v.at[c]], add=True)
        return carry

    lax.fori_loop(0, BPW // BLK, block_body, 0)
    plsc.subcore_barrier()

    _rows_copy(acc, p_out.at[cid], sid)


@functools.partial(
    pl.kernel,
    out_type=(
        jax.ShapeDtypeStruct((GP, F), jnp.float32),
        jax.ShapeDtypeStruct((GP, F), jnp.float32),
    ),
    mesh=_MESH,
    scratch_types=[
        pltpu.VMEM((NC * GACC * F,), jnp.float32),
        pltpu.VMEM((GP * 2,), jnp.int32),
        pltpu.VMEM((GP,), jnp.int32),
        pltpu.VMEM((GP,), jnp.int32),
        pltpu.VMEM((GP, F), jnp.float32),
        pltpu.VMEM((GP, F), jnp.float32),
        pltpu.SemaphoreType.DMA,
    ],
    compiler_params=pltpu.CompilerParams(needs_layout_passes=False),
)
def _head_kernel(h_hbm, cnt_hbm, set_hbm,
                 xsa_out, xsb_out, cnt_v, set_v, siba_v, sibb_v, ra_v, rb_v, sem):
    cid = lax.axis_index("c")
    sid = lax.axis_index("s")

    @pl.when(jnp.logical_and(cid == 0, sid == 0))
    def _():
        pltpu.sync_copy(cnt_hbm, cnt_v)
        pltpu.sync_copy(set_hbm, set_v)
        lane = lax.iota(jnp.int32, 16)
        carry = jnp.zeros((), jnp.int32)
        for c in range(GP // 16):
            rows = lane + c * 16
            cnt = (plsc.load_gather(cnt_v, [rows * F])
                   + plsc.load_gather(cnt_v, [rows * F + GACC * F]))
            cnt_i = cnt.astype(jnp.int32)
            bases = plsc.cumsum(cnt_i) - cnt_i + carry
            carry = carry + jnp.sum(cnt_i)
            sa = plsc.load_gather(set_v, [rows * 2])
            sb = plsc.load_gather(set_v, [rows * 2 + 1])
            siba_v[pl.ds(c * 16, 16)] = jnp.minimum(bases + sa, N - 1)
            sibb_v[pl.ds(c * 16, 16)] = jnp.minimum(bases + sb, N - 1)
        pltpu.async_copy(h_hbm.at[siba_v], ra_v, sem).wait()
        pltpu.async_copy(h_hbm.at[sibb_v], rb_v, sem).wait()
        pltpu.sync_copy(ra_v, xsa_out)
        pltpu.sync_copy(rb_v, xsb_out)


def _init_body(degp_ref, x_ref, sc_ref, g0_ref):
    d = degp_ref[0, :, 0:1] + degp_ref[1, :, 0:1] + 1.0
    dis = lax.rsqrt(d)
    v = 1.0 / d
    r = d * dis
    sc_ref[:, 0:1] = v
    sc_ref[:, 1:2] = dis
    sc_ref[:, 2:3] = r
    sc_ref[:, 3:8] = jnp.zeros_like(sc_ref[:, 3:8])
    g0_ref[...] = dis * x_ref[...]


def _tc_init(degp, x):
    return pl.pallas_call(
        _init_body,
        grid=(NBLK,),
        in_specs=[
            pl.BlockSpec((NC, ROWBLK, F), lambda i: (0, i, 0)),
            pl.BlockSpec((ROWBLK, F), lambda i: (i, 0)),
        ],
        out_specs=[
            pl.BlockSpec((ROWBLK, 8), lambda i: (i, 0)),
            pl.BlockSpec((ROWBLK, F), lambda i: (i, 0)),
        ],
        out_shape=[
            jax.ShapeDtypeStruct((N, 8), jnp.float32),
            jax.ShapeDtypeStruct((N, F), jnp.float32),
        ],
    )(degp, x)


def _hopmid_body(p_ref, sc_ref, g_ref):
    v = sc_ref[:, 0:1]
    g_ref[...] = v * (p_ref[0] + p_ref[1])


def _tc_hopmid(p, sc):
    return pl.pallas_call(
        _hopmid_body,
        grid=(NBLK,),
        in_specs=[
            pl.BlockSpec((NC, ROWBLK, F), lambda i: (0, i, 0)),
            pl.BlockSpec((ROWBLK, 8), lambda i: (i, 0)),
        ],
        out_specs=pl.BlockSpec((ROWBLK, F), lambda i: (i, 0)),
        out_shape=jax.ShapeDtypeStruct((N, F), jnp.float32),
    )(p, sc)


def _layer_body(g0_ref, g1_ref, g2_ref, g3_ref, sc_ref, w_ref, b_ref,
                lng_ref, lnb_ref, h_ref, gn_ref):
    acc = jnp.dot(g0_ref[...], w_ref[0:F, :], preferred_element_type=jnp.float32)
    acc += jnp.dot(g1_ref[...], w_ref[F:2 * F, :], preferred_element_type=jnp.float32)
    acc += jnp.dot(g2_ref[...], w_ref[2 * F:3 * F, :], preferred_element_type=jnp.float32)
    acc += jnp.dot(g3_ref[...], w_ref[3 * F:4 * F, :], preferred_element_type=jnp.float32)
    y = sc_ref[:, 2:3] * acc + b_ref[...]
    y = jnp.maximum(y, 0.0)
    mu = jnp.mean(y, axis=-1, keepdims=True)
    yc = y - mu
    var = jnp.mean(yc * yc, axis=-1, keepdims=True)
    h = lng_ref[...] * yc * lax.rsqrt(var + 1e-5) + lnb_ref[...]
    h_ref[...] = h
    gn_ref[...] = sc_ref[:, 1:2] * h


def _tc_layer(g0, g1, g2, g3, sc, wc, b, lng, lnb):
    return pl.pallas_call(
        _layer_body,
        grid=(NBLK,),
        in_specs=[
            pl.BlockSpec((ROWBLK, F), lambda i: (i, 0)),
            pl.BlockSpec((ROWBLK, F), lambda i: (i, 0)),
            pl.BlockSpec((ROWBLK, F), lambda i: (i, 0)),
            pl.BlockSpec((ROWBLK, F), lambda i: (i, 0)),
            pl.BlockSpec((ROWBLK, 8), lambda i: (i, 0)),
            pl.BlockSpec(((KHOPS + 1) * F, F), lambda i: (0, 0)),
            pl.BlockSpec((1, F), lambda i: (0, 0)),
            pl.BlockSpec((1, F), lambda i: (0, 0)),
            pl.BlockSpec((1, F), lambda i: (0, 0)),
        ],
        out_specs=[
            pl.BlockSpec((ROWBLK, F), lambda i: (i, 0)),
            pl.BlockSpec((ROWBLK, F), lambda i: (i, 0)),
        ],
        out_shape=[
            jax.ShapeDtypeStruct((N, F), jnp.float32),
            jax.ShapeDtypeStruct((N, F), jnp.float32),
        ],
    )(g0, g1, g2, g3, sc, wc, b, lng, lnb)


def _head_body(xsa_ref, xsb_ref, mw_ref, mb_ref, f1w_ref, f1b_ref,
               f2w_ref, f2b_ref, out_ref):
    a = xsa_ref[...]
    b = xsb_ref[...]
    df = jnp.abs(a - b)
    mn = 0.5 * (a + b)
    mx = jnp.maximum(a, b)
    m = jnp.dot(df, mw_ref[0:F, :], preferred_element_type=jnp.float32)
    m += jnp.dot(mn, mw_ref[F:2 * F, :], preferred_element_type=jnp.float32)
    m += jnp.dot(mx, mw_ref[2 * F:3 * F, :], preferred_element_type=jnp.float32)
    m += mb_ref[...]
    f = jnp.maximum(jnp.dot(m, f1w_ref[...], preferred_element_type=jnp.float32)
                    + f1b_ref[...], 0.0)
    o = jnp.dot(f, f2w_ref[...], preferred_element_type=jnp.float32) + f2b_ref[...]
    out_ref[...] = o[0:G, :]


def _tc_head(xsa, xsb, mw, mb, f1w, f1b, f2w, f2b):
    return pl.pallas_call(
        _head_body,
        grid=(1,),
        in_specs=[
            pl.BlockSpec((GP, F), lambda i: (0, 0)),
            pl.BlockSpec((GP, F), lambda i: (0, 0)),
            pl.BlockSpec((3 * F, F), lambda i: (0, 0)),
            pl.BlockSpec((1, F), lambda i: (0, 0)),
            pl.BlockSpec((F, F), lambda i: (0, 0)),
            pl.BlockSpec((1, F), lambda i: (0, 0)),
            pl.BlockSpec((F, F), lambda i: (0, 0)),
            pl.BlockSpec((1, F), lambda i: (0, 0)),
        ],
        out_specs=pl.BlockSpec((G, F), lambda i: (0, 0)),
        out_shape=jax.ShapeDtypeStruct((G, F), jnp.float32),
    )(xsa, xsb, mw, mb, f1w, f1b, f2w, f2b)


def kernel(x, edge_index, set_indices, batch_ids, num_graphs, W0, b0, W1, b1,
           ln0_g, ln0_b, ln1_g, ln1_b, merger_W, merger_b, ff1_W, ff1_b,
           ff2_W, ff2_b):
    src = edge_index[0]
    dst = edge_index[1]
    epad = E2 - E
    src_pad = jnp.concatenate(
        [src, (jnp.arange(epad, dtype=jnp.int32) * 67) % N]).reshape(E2 // CH, CH)
    dst_pad = jnp.concatenate(
        [dst, N + (jnp.arange(epad, dtype=jnp.int32) % CH)]).reshape(E2 // CH, CH)
    bids_pad = jnp.concatenate(
        [batch_ids, jnp.full((BP - N,), GACC - 1, jnp.int32)])
    set_pad = jnp.concatenate(
        [set_indices, jnp.zeros((GP - G, 2), jnp.int32)], axis=0)
    zeros_nf = jnp.zeros((N, F), jnp.float32)
    ones_chf = jnp.ones((CH, F), jnp.float32)

    degp, cntp = _deg_kernel(dst, bids_pad, ones_chf, zeros_nf)
    sc, g0 = _tc_init(degp, x)

    def run_layer(g_first, wc, b, lng, lnb):
        gs = [g_first]
        for _ in range(KHOPS):
            p = _hop_kernel(gs[-1], src_pad, dst_pad, zeros_nf)
            gs.append(_tc_hopmid(p, sc))
        return _tc_layer(gs[0], gs[1], gs[2], gs[3], sc, wc,
                         b.reshape(1, F), lng.reshape(1, F), lnb.reshape(1, F))

    wc0 = W0.reshape((KHOPS + 1) * F, F)
    wc1 = W1.reshape((KHOPS + 1) * F, F)
    h1, g1_first = run_layer(g0, wc0, b0, ln0_g, ln0_b)
    h2, _ = run_layer(g1_first, wc1, b1, ln1_g, ln1_b)

    xsa, xsb = _head_kernel(h2, cntp.reshape(-1), set_pad.reshape(-1))
    out = _tc_head(xsa, xsb, merger_W, merger_b.reshape(1, F),
                   ff1_W, ff1_b.reshape(1, F), ff2_W, ff2_b.reshape(1, F))
    return out


# speedup vs baseline: 3.3458x; 1.4216x over previous
import functools

import jax
import jax.numpy as jnp
from jax import lax
from jax.experimental import pallas as pl
from jax.experimental.pallas import tpu as pltpu
from jax.experimental.pallas import tpu_sc as plsc

N = 10000
E = 320000
F = 128
KHOPS = 3
G = 100
GP = 112
GACC = 128
NC = 2
NS = 16
NW = NC * NS
CH = 128
NCHUNK = E // CH
RA = 624
RL = N - (NS - 1) * RA
BP = 10240
NBCH = BP // CH
BLK = 1024
BPW = 10240
E2 = NW * BPW
ROWBLK = 1000
NBLK = N // ROWBLK

_MESH = plsc.VectorSubcoreMesh(core_axis_name="c", subcore_axis_name="s",
                               num_cores=NC, num_subcores=NS)


def _rows_copy(src, dst, sid):
    off = pl.multiple_of(sid * RA, 8)

    @pl.when(sid < NS - 1)
    def _():
        pltpu.sync_copy(src.at[pl.ds(off, RA)], dst.at[pl.ds(off, RA)])

    @pl.when(sid == NS - 1)
    def _():
        pltpu.sync_copy(src.at[pl.ds((NS - 1) * RA, RL)],
                        dst.at[pl.ds((NS - 1) * RA, RL)])


def _chunks_for(w, total):
    full = total // NW
    extra = total - full * NW
    return jnp.where(w < extra, full + 1, full)


@functools.partial(
    pl.kernel,
    out_type=(
        jax.ShapeDtypeStruct((NC, N, F), jnp.float32),
        jax.ShapeDtypeStruct((NC, GACC, F), jnp.float32),
    ),
    mesh=_MESH,
    scratch_types=[
        pltpu.VMEM_SHARED((N, F), jnp.float32),
        pltpu.VMEM_SHARED((GACC, F), jnp.float32),
        pltpu.VMEM((1, CH), jnp.int32),
        pltpu.VMEM((CH, F), jnp.float32),
    ],
)
def _deg_kernel(dst_hbm, bids_hbm, ones_hbm, zeros_hbm,
                deg_out, cnt_out, acc_deg, acc_cnt, idx_v, ones_v):
    cid = lax.axis_index("c")
    sid = lax.axis_index("s")
    w = cid * NS + sid

    _rows_copy(zeros_hbm, acc_deg, sid)

    @pl.when(sid == 0)
    def _():
        pltpu.sync_copy(zeros_hbm.at[pl.ds(0, GACC)], acc_cnt)

    pltpu.sync_copy(ones_hbm, ones_v)
    plsc.subcore_barrier()

    def edge_body(j, _):
        ci = w + j * NW
        off = pl.multiple_of(ci * CH, CH)
        pltpu.sync_copy(dst_hbm.at[pl.ds(off, CH)], idx_v.at[0])
        pltpu.sync_copy(ones_v, acc_deg.at[idx_v.at[0]], add=True)
        return _

    lax.fori_loop(0, _chunks_for(w, NCHUNK), edge_body, 0)

    def bid_body(j, _):
        ci = w + j * NW
        off = pl.multiple_of(ci * CH, CH)
        pltpu.sync_copy(bids_hbm.at[pl.ds(off, CH)], idx_v.at[0])
        pltpu.sync_copy(ones_v, acc_cnt.at[idx_v.at[0]], add=True)
        return _

    lax.fori_loop(0, _chunks_for(w, NBCH), bid_body, 0)
    plsc.subcore_barrier()

    _rows_copy(acc_deg, deg_out.at[cid], sid)

    @pl.when(sid == 0)
    def _():
        pltpu.sync_copy(acc_cnt, cnt_out.at[cid])


@functools.partial(
    pl.kernel,
    out_type=jax.ShapeDtypeStruct((NC, N, F), jnp.float32),
    mesh=_MESH,
    scratch_types=[
        pltpu.VMEM_SHARED((N + CH, F), jnp.float32),
        pltpu.VMEM((CH,), jnp.int32),
        pltpu.VMEM((CH,), jnp.int32),
        pltpu.VMEM((BLK // CH, CH), jnp.int32),
        pltpu.VMEM((CH, F), jnp.float32),
        pltpu.VMEM((CH, F), jnp.float32),
        pltpu.SemaphoreType.DMA,
        pltpu.SemaphoreType.DMA,
    ],
)
def _hop_kernel(g_hbm, src_hbm, dst_hbm, zeros_hbm,
                p_out, acc, sidx0, sidx1, didx_v, rows0, rows1, sem0, sem1):
    cid = lax.axis_index("c")
    sid = lax.axis_index("s")
    w = cid * NS + sid

    @pl.when(cid == 0)
    def _():
        _rows_copy(g_hbm, acc, sid)

    @pl.when(cid != 0)
    def _():
        _rows_copy(zeros_hbm, acc, sid)

    plsc.subcore_barrier()

    bufs = (rows0, rows1)
    sems = (sem0, sem1)
    nch = BLK // CH

    sidx = (sidx0, sidx1)

    def block_body(b, carry):
        brow = pl.multiple_of(w * (BPW // CH) + b * nch, 8)
        pltpu.sync_copy(dst_hbm.at[pl.ds(brow, nch)], didx_v)
        descs = [None, None]
        pltpu.sync_copy(src_hbm.at[brow], sidx[0])
        descs[0] = pltpu.async_copy(g_hbm.at[sidx[0]], bufs[0], sems[0])
        for c in range(nch):
            if c < nch - 1:
                nb = (c + 1) % 2
                pltpu.sync_copy(src_hbm.at[brow + c + 1], sidx[nb])
                descs[nb] = pltpu.async_copy(g_hbm.at[sidx[nb]], bufs[nb], sems[nb])
            descs[c % 2].wait()
            pltpu.sync_copy(bufs[c % 2], acc.at[didx_v.at[c]], add=True)
        return carry

    lax.fori_loop(0, BPW // BLK, block_body, 0)
    plsc.subcore_barrier()

    _rows_copy(acc, p_out.at[cid], sid)


@functools.partial(
    pl.kernel,
    out_type=(
        jax.ShapeDtypeStruct((GP, F), jnp.float32),
        jax.ShapeDtypeStruct((GP, F), jnp.float32),
    ),
    mesh=_MESH,
    scratch_types=[
        pltpu.VMEM((NC * GACC * F,), jnp.float32),
        pltpu.VMEM((GP * 2,), jnp.int32),
        pltpu.VMEM((GP,), jnp.int32),
        pltpu.VMEM((GP,), jnp.int32),
        pltpu.VMEM((GP, F), jnp.float32),
        pltpu.VMEM((GP, F), jnp.float32),
        pltpu.SemaphoreType.DMA,
    ],
    compiler_params=pltpu.CompilerParams(needs_layout_passes=False),
)
def _head_kernel(h_hbm, cnt_hbm, set_hbm,
                 xsa_out, xsb_out, cnt_v, set_v, siba_v, sibb_v, ra_v, rb_v, sem):
    cid = lax.axis_index("c")
    sid = lax.axis_index("s")

    @pl.when(jnp.logical_and(cid == 0, sid == 0))
    def _():
        pltpu.sync_copy(cnt_hbm, cnt_v)
        pltpu.sync_copy(set_hbm, set_v)
        lane = lax.iota(jnp.int32, 16)
        carry = jnp.zeros((), jnp.int32)
        for c in range(GP // 16):
            rows = lane + c * 16
            cnt = (plsc.load_gather(cnt_v, [rows * F])
                   + plsc.load_gather(cnt_v, [rows * F + GACC * F]))
            cnt_i = cnt.astype(jnp.int32)
            bases = plsc.cumsum(cnt_i) - cnt_i + carry
            carry = carry + jnp.sum(cnt_i)
            sa = plsc.load_gather(set_v, [rows * 2])
            sb = plsc.load_gather(set_v, [rows * 2 + 1])
            siba_v[pl.ds(c * 16, 16)] = jnp.minimum(bases + sa, N - 1)
            sibb_v[pl.ds(c * 16, 16)] = jnp.minimum(bases + sb, N - 1)
        pltpu.async_copy(h_hbm.at[siba_v], ra_v, sem).wait()
        pltpu.async_copy(h_hbm.at[sibb_v], rb_v, sem).wait()
        pltpu.sync_copy(ra_v, xsa_out)
        pltpu.sync_copy(rb_v, xsb_out)


def _init_body(degp_ref, x_ref, sc_ref, g0_ref):
    d = degp_ref[0, :, 0:1] + degp_ref[1, :, 0:1] + 1.0
    dis = lax.rsqrt(d)
    v = 1.0 / d
    r = d * dis
    sc_ref[:, 0:1] = v
    sc_ref[:, 1:2] = dis
    sc_ref[:, 2:3] = r
    sc_ref[:, 3:8] = jnp.zeros_like(sc_ref[:, 3:8])
    g0_ref[...] = dis * x_ref[...]


def _tc_init(degp, x):
    return pl.pallas_call(
        _init_body,
        grid=(NBLK,),
        in_specs=[
            pl.BlockSpec((NC, ROWBLK, F), lambda i: (0, i, 0)),
            pl.BlockSpec((ROWBLK, F), lambda i: (i, 0)),
        ],
        out_specs=[
            pl.BlockSpec((ROWBLK, 8), lambda i: (i, 0)),
            pl.BlockSpec((ROWBLK, F), lambda i: (i, 0)),
        ],
        out_shape=[
            jax.ShapeDtypeStruct((N, 8), jnp.float32),
            jax.ShapeDtypeStruct((N, F), jnp.float32),
        ],
    )(degp, x)


def _hopmid_body(p_ref, sc_ref, g_ref):
    v = sc_ref[:, 0:1]
    g_ref[...] = v * (p_ref[0] + p_ref[1])


def _tc_hopmid(p, sc):
    return pl.pallas_call(
        _hopmid_body,
        grid=(NBLK,),
        in_specs=[
            pl.BlockSpec((NC, ROWBLK, F), lambda i: (0, i, 0)),
            pl.BlockSpec((ROWBLK, 8), lambda i: (i, 0)),
        ],
        out_specs=pl.BlockSpec((ROWBLK, F), lambda i: (i, 0)),
        out_shape=jax.ShapeDtypeStruct((N, F), jnp.float32),
    )(p, sc)


def _layer_body(g0_ref, g1_ref, g2_ref, g3_ref, sc_ref, w_ref, b_ref,
                lng_ref, lnb_ref, h_ref, gn_ref):
    acc = jnp.dot(g0_ref[...], w_ref[0:F, :], preferred_element_type=jnp.float32)
    acc += jnp.dot(g1_ref[...], w_ref[F:2 * F, :], preferred_element_type=jnp.float32)
    acc += jnp.dot(g2_ref[...], w_ref[2 * F:3 * F, :], preferred_element_type=jnp.float32)
    acc += jnp.dot(g3_ref[...], w_ref[3 * F:4 * F, :], preferred_element_type=jnp.float32)
    y = sc_ref[:, 2:3] * acc + b_ref[...]
    y = jnp.maximum(y, 0.0)
    mu = jnp.mean(y, axis=-1, keepdims=True)
    yc = y - mu
    var = jnp.mean(yc * yc, axis=-1, keepdims=True)
    h = lng_ref[...] * yc * lax.rsqrt(var + 1e-5) + lnb_ref[...]
    h_ref[...] = h
    gn_ref[...] = sc_ref[:, 1:2] * h


def _tc_layer(g0, g1, g2, g3, sc, wc, b, lng, lnb):
    return pl.pallas_call(
        _layer_body,
        grid=(NBLK,),
        in_specs=[
            pl.BlockSpec((ROWBLK, F), lambda i: (i, 0)),
            pl.BlockSpec((ROWBLK, F), lambda i: (i, 0)),
            pl.BlockSpec((ROWBLK, F), lambda i: (i, 0)),
            pl.BlockSpec((ROWBLK, F), lambda i: (i, 0)),
            pl.BlockSpec((ROWBLK, 8), lambda i: (i, 0)),
            pl.BlockSpec(((KHOPS + 1) * F, F), lambda i: (0, 0)),
            pl.BlockSpec((1, F), lambda i: (0, 0)),
            pl.BlockSpec((1, F), lambda i: (0, 0)),
            pl.BlockSpec((1, F), lambda i: (0, 0)),
        ],
        out_specs=[
            pl.BlockSpec((ROWBLK, F), lambda i: (i, 0)),
            pl.BlockSpec((ROWBLK, F), lambda i: (i, 0)),
        ],
        out_shape=[
            jax.ShapeDtypeStruct((N, F), jnp.float32),
            jax.ShapeDtypeStruct((N, F), jnp.float32),
        ],
    )(g0, g1, g2, g3, sc, wc, b, lng, lnb)


def _head_body(xsa_ref, xsb_ref, mw_ref, mb_ref, f1w_ref, f1b_ref,
               f2w_ref, f2b_ref, out_ref):
    a = xsa_ref[...]
    b = xsb_ref[...]
    df = jnp.abs(a - b)
    mn = 0.5 * (a + b)
    mx = jnp.maximum(a, b)
    m = jnp.dot(df, mw_ref[0:F, :], preferred_element_type=jnp.float32)
    m += jnp.dot(mn, mw_ref[F:2 * F, :], preferred_element_type=jnp.float32)
    m += jnp.dot(mx, mw_ref[2 * F:3 * F, :], preferred_element_type=jnp.float32)
    m += mb_ref[...]
    f = jnp.maximum(jnp.dot(m, f1w_ref[...], preferred_element_type=jnp.float32)
                    + f1b_ref[...], 0.0)
    o = jnp.dot(f, f2w_ref[...], preferred_element_type=jnp.float32) + f2b_ref[...]
    out_ref[...] = o[0:G, :]


def _tc_head(xsa, xsb, mw, mb, f1w, f1b, f2w, f2b):
    return pl.pallas_call(
        _head_body,
        grid=(1,),
        in_specs=[
            pl.BlockSpec((GP, F), lambda i: (0, 0)),
            pl.BlockSpec((GP, F), lambda i: (0, 0)),
            pl.BlockSpec((3 * F, F), lambda i: (0, 0)),
            pl.BlockSpec((1, F), lambda i: (0, 0)),
            pl.BlockSpec((F, F), lambda i: (0, 0)),
            pl.BlockSpec((1, F), lambda i: (0, 0)),
            pl.BlockSpec((F, F), lambda i: (0, 0)),
            pl.BlockSpec((1, F), lambda i: (0, 0)),
        ],
        out_specs=pl.BlockSpec((G, F), lambda i: (0, 0)),
        out_shape=jax.ShapeDtypeStruct((G, F), jnp.float32),
    )(xsa, xsb, mw, mb, f1w, f1b, f2w, f2b)


def kernel(x, edge_index, set_indices, batch_ids, num_graphs, W0, b0, W1, b1,
           ln0_g, ln0_b, ln1_g, ln1_b, merger_W, merger_b, ff1_W, ff1_b,
           ff2_W, ff2_b):
    src = edge_index[0]
    dst = edge_index[1]
    epad = E2 - E
    src_pad = jnp.concatenate(
        [src, (jnp.arange(epad, dtype=jnp.int32) * 67) % N]).reshape(E2 // CH, CH)
    dst_pad = jnp.concatenate(
        [dst, N + (jnp.arange(epad, dtype=jnp.int32) % CH)]).reshape(E2 // CH, CH)
    bids_pad = jnp.concatenate(
        [batch_ids, jnp.full((BP - N,), GACC - 1, jnp.int32)])
    set_pad = jnp.concatenate(
        [set_indices, jnp.zeros((GP - G, 2), jnp.int32)], axis=0)
    zeros_nf = jnp.zeros((N, F), jnp.float32)
    ones_chf = jnp.ones((CH, F), jnp.float32)

    degp, cntp = _deg_kernel(dst, bids_pad, ones_chf, zeros_nf)
    sc, g0 = _tc_init(degp, x)

    def run_layer(g_first, wc, b, lng, lnb):
        gs = [g_first]
        for _ in range(KHOPS):
            p = _hop_kernel(gs[-1], src_pad, dst_pad, zeros_nf)
            gs.append(_tc_hopmid(p, sc))
        return _tc_layer(gs[0], gs[1], gs[2], gs[3], sc, wc,
                         b.reshape(1, F), lng.reshape(1, F), lnb.reshape(1, F))

    wc0 = W0.reshape((KHOPS + 1) * F, F)
    wc1 = W1.reshape((KHOPS + 1) * F, F)
    h1, g1_first = run_layer(g0, wc0, b0, ln0_g, ln0_b)
    h2, _ = run_layer(g1_first, wc1, b1, ln1_g, ln1_b)

    xsa, xsb = _head_kernel(h2, cntp.reshape(-1), set_pad.reshape(-1))
    out = _tc_head(xsa, xsb, merger_W, merger_b.reshape(1, F),
                   ff1_W, ff1_b.reshape(1, F), ff2_W, ff2_b.reshape(1, F))
    return out


# speedup vs baseline: 3.6112x; 1.0793x over previous
import functools

import jax
import jax.numpy as jnp
from jax import lax
from jax.experimental import pallas as pl
from jax.experimental.pallas import tpu as pltpu
from jax.experimental.pallas import tpu_sc as plsc

N = 10000
E = 320000
F = 128
KHOPS = 3
G = 100
GP = 112
GACC = 128
NC = 2
NS = 16
NW = NC * NS
CH = 128
NCHUNK = E // CH
RA = 624
RL = N - (NS - 1) * RA
BP = 10240
NBCH = BP // CH
BLK = 2048
NBUF = 4
PDEP = 2
BPW = 10240
E2 = NW * BPW
ROWBLK = 1000
NBLK = N // ROWBLK

_MESH = plsc.VectorSubcoreMesh(core_axis_name="c", subcore_axis_name="s",
                               num_cores=NC, num_subcores=NS)


def _rows_copy(src, dst, sid):
    off = pl.multiple_of(sid * RA, 8)

    @pl.when(sid < NS - 1)
    def _():
        pltpu.sync_copy(src.at[pl.ds(off, RA)], dst.at[pl.ds(off, RA)])

    @pl.when(sid == NS - 1)
    def _():
        pltpu.sync_copy(src.at[pl.ds((NS - 1) * RA, RL)],
                        dst.at[pl.ds((NS - 1) * RA, RL)])


def _chunks_for(w, total):
    full = total // NW
    extra = total - full * NW
    return jnp.where(w < extra, full + 1, full)


@functools.partial(
    pl.kernel,
    out_type=(
        jax.ShapeDtypeStruct((NC, N, F), jnp.float32),
        jax.ShapeDtypeStruct((NC, GACC, F), jnp.float32),
    ),
    mesh=_MESH,
    scratch_types=[
        pltpu.VMEM_SHARED((N + CH, F), jnp.float32),
        pltpu.VMEM_SHARED((GACC, F), jnp.float32),
        pltpu.VMEM((BLK // CH, CH), jnp.int32),
        pltpu.VMEM((1, CH), jnp.int32),
        pltpu.VMEM((CH, F), jnp.float32),
        pltpu.SemaphoreType.DMA,
        pltpu.SemaphoreType.DMA,
        pltpu.SemaphoreType.DMA,
        pltpu.SemaphoreType.DMA,
    ],
)
def _deg_kernel(dst_hbm, bids_hbm, ones_hbm, zeros_hbm,
                deg_out, cnt_out, acc_deg, acc_cnt, didx_v, idx_v, ones_v,
                t0, t1, t2, t3):
    cid = lax.axis_index("c")
    sid = lax.axis_index("s")
    w = cid * NS + sid
    ss = (t0, t1, t2, t3)
    nch = BLK // CH

    _rows_copy(zeros_hbm, acc_deg, sid)

    @pl.when(sid == 0)
    def _():
        pltpu.sync_copy(zeros_hbm.at[pl.ds(0, GACC)], acc_cnt)

    pltpu.sync_copy(ones_hbm, ones_v)
    plsc.subcore_barrier()

    def block_body(b, carry):
        brow = pl.multiple_of(w * (BPW // CH) + b * nch, 8)
        pltpu.sync_copy(dst_hbm.at[pl.ds(brow, nch)], didx_v)
        ds = [None] * NBUF
        for c in range(nch):
            if c >= NBUF:
                ds[c % NBUF].wait()
            ds[c % NBUF] = pltpu.async_copy(
                ones_v, acc_deg.at[didx_v.at[c]], ss[c % NBUF], add=True)
        for j in range(nch - NBUF, nch):
            ds[j % NBUF].wait()
        return carry

    lax.fori_loop(0, BPW // BLK, block_body, 0)

    def bid_body(j, _):
        ci = w + j * NW
        off = pl.multiple_of(ci * CH, CH)
        pltpu.sync_copy(bids_hbm.at[pl.ds(off, CH)], idx_v.at[0])
        pltpu.sync_copy(ones_v, acc_cnt.at[idx_v.at[0]], add=True)
        return _

    lax.fori_loop(0, _chunks_for(w, NBCH), bid_body, 0)
    plsc.subcore_barrier()

    _rows_copy(acc_deg, deg_out.at[cid], sid)

    @pl.when(sid == 0)
    def _():
        pltpu.sync_copy(acc_cnt, cnt_out.at[cid])


@functools.partial(
    pl.kernel,
    out_type=jax.ShapeDtypeStruct((NC, N, F), jnp.float32),
    mesh=_MESH,
    scratch_types=(
        [pltpu.VMEM_SHARED((N + CH, F), jnp.float32)]
        + [pltpu.VMEM((CH,), jnp.int32)] * 2
        + [pltpu.VMEM((BLK // CH, CH), jnp.int32)]
        + [pltpu.VMEM((CH, F), jnp.float32)] * 2
        + [pltpu.SemaphoreType.DMA] * 4
    ),
)
def _hop_kernel(g_hbm, src_hbm, dst_hbm, zeros_hbm, p_out, acc,
                s0, s1, didx_v, r0, r1, g0, g1, t0, t1):
    cid = lax.axis_index("c")
    sid = lax.axis_index("s")
    w = cid * NS + sid
    sidx = (s0, s1)
    bufs = (r0, r1)
    sg = (g0, g1)
    ss = (t0, t1)
    nch = BLK // CH

    @pl.when(cid == 0)
    def _():
        _rows_copy(g_hbm, acc, sid)

    @pl.when(cid != 0)
    def _():
        _rows_copy(zeros_hbm, acc, sid)

    plsc.subcore_barrier()

    def block_body(b, carry):
        brow = pl.multiple_of(w * (BPW // CH) + b * nch, 8)
        pltpu.sync_copy(dst_hbm.at[pl.ds(brow, nch)], didx_v)
        dg = [None, None]
        ds = [None, None]
        for c in range(nch + 1):
            if c < nch:
                i = c % 2
                if c >= 2:
                    ds[i].wait()
                pltpu.sync_copy(src_hbm.at[brow + c], sidx[i])
                dg[i] = pltpu.async_copy(g_hbm.at[sidx[i]], bufs[i], sg[i])
            if c >= 1:
                j = c - 1
                i = j % 2
                dg[i].wait()
                ds[i] = pltpu.async_copy(
                    bufs[i], acc.at[didx_v.at[j]], ss[i], add=True)
        for j in range(nch - 2, nch):
            ds[j % 2].wait()
        return carry

    lax.fori_loop(0, BPW // BLK, block_body, 0)
    plsc.subcore_barrier()

    _rows_copy(acc, p_out.at[cid], sid)


@functools.partial(
    pl.kernel,
    out_type=(
        jax.ShapeDtypeStruct((GP, F), jnp.float32),
        jax.ShapeDtypeStruct((GP, F), jnp.float32),
    ),
    mesh=_MESH,
    scratch_types=[
        pltpu.VMEM((NC * GACC * F,), jnp.float32),
        pltpu.VMEM((GP * 2,), jnp.int32),
        pltpu.VMEM((GP,), jnp.int32),
        pltpu.VMEM((GP,), jnp.int32),
        pltpu.VMEM((GP, F), jnp.float32),
        pltpu.VMEM((GP, F), jnp.float32),
        pltpu.SemaphoreType.DMA,
    ],
    compiler_params=pltpu.CompilerParams(needs_layout_passes=False),
)
def _head_kernel(h_hbm, cnt_hbm, set_hbm,
                 xsa_out, xsb_out, cnt_v, set_v, siba_v, sibb_v, ra_v, rb_v, sem):
    cid = lax.axis_index("c")
    sid = lax.axis_index("s")

    @pl.when(jnp.logical_and(cid == 0, sid == 0))
    def _():
        pltpu.sync_copy(cnt_hbm, cnt_v)
        pltpu.sync_copy(set_hbm, set_v)
        lane = lax.iota(jnp.int32, 16)
        carry = jnp.zeros((), jnp.int32)
        for c in range(GP // 16):
            rows = lane + c * 16
            cnt = (plsc.load_gather(cnt_v, [rows * F])
                   + plsc.load_gather(cnt_v, [rows * F + GACC * F]))
            cnt_i = cnt.astype(jnp.int32)
            bases = plsc.cumsum(cnt_i) - cnt_i + carry
            carry = carry + jnp.sum(cnt_i)
            sa = plsc.load_gather(set_v, [rows * 2])
            sb = plsc.load_gather(set_v, [rows * 2 + 1])
            siba_v[pl.ds(c * 16, 16)] = jnp.minimum(bases + sa, N - 1)
            sibb_v[pl.ds(c * 16, 16)] = jnp.minimum(bases + sb, N - 1)
        pltpu.async_copy(h_hbm.at[siba_v], ra_v, sem).wait()
        pltpu.async_copy(h_hbm.at[sibb_v], rb_v, sem).wait()
        pltpu.sync_copy(ra_v, xsa_out)
        pltpu.sync_copy(rb_v, xsb_out)


def _init_body(degp_ref, x_ref, sc_ref, g0_ref):
    d = degp_ref[0, :, 0:1] + degp_ref[1, :, 0:1] + 1.0
    dis = lax.rsqrt(d)
    v = 1.0 / d
    r = d * dis
    sc_ref[:, 0:1] = v
    sc_ref[:, 1:2] = dis
    sc_ref[:, 2:3] = r
    sc_ref[:, 3:8] = jnp.zeros_like(sc_ref[:, 3:8])
    g0_ref[...] = dis * x_ref[...]


def _tc_init(degp, x):
    return pl.pallas_call(
        _init_body,
        grid=(NBLK,),
        in_specs=[
            pl.BlockSpec((NC, ROWBLK, F), lambda i: (0, i, 0)),
            pl.BlockSpec((ROWBLK, F), lambda i: (i, 0)),
        ],
        out_specs=[
            pl.BlockSpec((ROWBLK, 8), lambda i: (i, 0)),
            pl.BlockSpec((ROWBLK, F), lambda i: (i, 0)),
        ],
        out_shape=[
            jax.ShapeDtypeStruct((N, 8), jnp.float32),
            jax.ShapeDtypeStruct((N, F), jnp.float32),
        ],
    )(degp, x)


def _hopmid_body(p_ref, sc_ref, g_ref):
    v = sc_ref[:, 0:1]
    g_ref[...] = v * (p_ref[0] + p_ref[1])


def _tc_hopmid(p, sc):
    return pl.pallas_call(
        _hopmid_body,
        grid=(NBLK,),
        in_specs=[
            pl.BlockSpec((NC, ROWBLK, F), lambda i: (0, i, 0)),
            pl.BlockSpec((ROWBLK, 8), lambda i: (i, 0)),
        ],
        out_specs=pl.BlockSpec((ROWBLK, F), lambda i: (i, 0)),
        out_shape=jax.ShapeDtypeStruct((N, F), jnp.float32),
    )(p, sc)


def _layer_body(g0_ref, g1_ref, g2_ref, g3_ref, sc_ref, w_ref, b_ref,
                lng_ref, lnb_ref, h_ref, gn_ref):
    acc = jnp.dot(g0_ref[...], w_ref[0:F, :], preferred_element_type=jnp.float32)
    acc += jnp.dot(g1_ref[...], w_ref[F:2 * F, :], preferred_element_type=jnp.float32)
    acc += jnp.dot(g2_ref[...], w_ref[2 * F:3 * F, :], preferred_element_type=jnp.float32)
    acc += jnp.dot(g3_ref[...], w_ref[3 * F:4 * F, :], preferred_element_type=jnp.float32)
    y = sc_ref[:, 2:3] * acc + b_ref[...]
    y = jnp.maximum(y, 0.0)
    mu = jnp.mean(y, axis=-1, keepdims=True)
    yc = y - mu
    var = jnp.mean(yc * yc, axis=-1, keepdims=True)
    h = lng_ref[...] * yc * lax.rsqrt(var + 1e-5) + lnb_ref[...]
    h_ref[...] = h
    gn_ref[...] = sc_ref[:, 1:2] * h


def _tc_layer(g0, g1, g2, g3, sc, wc, b, lng, lnb):
    return pl.pallas_call(
        _layer_body,
        grid=(NBLK,),
        in_specs=[
            pl.BlockSpec((ROWBLK, F), lambda i: (i, 0)),
            pl.BlockSpec((ROWBLK, F), lambda i: (i, 0)),
            pl.BlockSpec((ROWBLK, F), lambda i: (i, 0)),
            pl.BlockSpec((ROWBLK, F), lambda i: (i, 0)),
            pl.BlockSpec((ROWBLK, 8), lambda i: (i, 0)),
            pl.BlockSpec(((KHOPS + 1) * F, F), lambda i: (0, 0)),
            pl.BlockSpec((1, F), lambda i: (0, 0)),
            pl.BlockSpec((1, F), lambda i: (0, 0)),
            pl.BlockSpec((1, F), lambda i: (0, 0)),
        ],
        out_specs=[
            pl.BlockSpec((ROWBLK, F), lambda i: (i, 0)),
            pl.BlockSpec((ROWBLK, F), lambda i: (i, 0)),
        ],
        out_shape=[
            jax.ShapeDtypeStruct((N, F), jnp.float32),
            jax.ShapeDtypeStruct((N, F), jnp.float32),
        ],
    )(g0, g1, g2, g3, sc, wc, b, lng, lnb)


def _head_body(xsa_ref, xsb_ref, mw_ref, mb_ref, f1w_ref, f1b_ref,
               f2w_ref, f2b_ref, out_ref):
    a = xsa_ref[...]
    b = xsb_ref[...]
    df = jnp.abs(a - b)
    mn = 0.5 * (a + b)
    mx = jnp.maximum(a, b)
    m = jnp.dot(df, mw_ref[0:F, :], preferred_element_type=jnp.float32)
    m += jnp.dot(mn, mw_ref[F:2 * F, :], preferred_element_type=jnp.float32)
    m += jnp.dot(mx, mw_ref[2 * F:3 * F, :], preferred_element_type=jnp.float32)
    m += mb_ref[...]
    f = jnp.maximum(jnp.dot(m, f1w_ref[...], preferred_element_type=jnp.float32)
                    + f1b_ref[...], 0.0)
    o = jnp.dot(f, f2w_ref[...], preferred_element_type=jnp.float32) + f2b_ref[...]
    out_ref[...] = o[0:G, :]


def _tc_head(xsa, xsb, mw, mb, f1w, f1b, f2w, f2b):
    return pl.pallas_call(
        _head_body,
        grid=(1,),
        in_specs=[
            pl.BlockSpec((GP, F), lambda i: (0, 0)),
            pl.BlockSpec((GP, F), lambda i: (0, 0)),
            pl.BlockSpec((3 * F, F), lambda i: (0, 0)),
            pl.BlockSpec((1, F), lambda i: (0, 0)),
            pl.BlockSpec((F, F), lambda i: (0, 0)),
            pl.BlockSpec((1, F), lambda i: (0, 0)),
            pl.BlockSpec((F, F), lambda i: (0, 0)),
            pl.BlockSpec((1, F), lambda i: (0, 0)),
        ],
        out_specs=pl.BlockSpec((G, F), lambda i: (0, 0)),
        out_shape=jax.ShapeDtypeStruct((G, F), jnp.float32),
    )(xsa, xsb, mw, mb, f1w, f1b, f2w, f2b)


def kernel(x, edge_index, set_indices, batch_ids, num_graphs, W0, b0, W1, b1,
           ln0_g, ln0_b, ln1_g, ln1_b, merger_W, merger_b, ff1_W, ff1_b,
           ff2_W, ff2_b):
    src = edge_index[0]
    dst = edge_index[1]
    epad = E2 - E
    src_pad = jnp.concatenate(
        [src, (jnp.arange(epad, dtype=jnp.int32) * 67) % N]).reshape(E2 // CH, CH)
    dst_pad = jnp.concatenate(
        [dst, N + (jnp.arange(epad, dtype=jnp.int32) % CH)]).reshape(E2 // CH, CH)
    bids_pad = jnp.concatenate(
        [batch_ids, jnp.full((BP - N,), GACC - 1, jnp.int32)])
    set_pad = jnp.concatenate(
        [set_indices, jnp.zeros((GP - G, 2), jnp.int32)], axis=0)
    zeros_nf = jnp.zeros((N, F), jnp.float32)
    ones_chf = jnp.ones((CH, F), jnp.float32)

    degp, cntp = _deg_kernel(dst_pad, bids_pad, ones_chf, zeros_nf)
    sc, g0 = _tc_init(degp, x)

    def run_layer(g_first, wc, b, lng, lnb):
        gs = [g_first]
        for _ in range(KHOPS):
            p = _hop_kernel(gs[-1], src_pad, dst_pad, zeros_nf)
            gs.append(_tc_hopmid(p, sc))
        return _tc_layer(gs[0], gs[1], gs[2], gs[3], sc, wc,
                         b.reshape(1, F), lng.reshape(1, F), lnb.reshape(1, F))

    wc0 = W0.reshape((KHOPS + 1) * F, F)
    wc1 = W1.reshape((KHOPS + 1) * F, F)
    h1, g1_first = run_layer(g0, wc0, b0, ln0_g, ln0_b)
    h2, _ = run_layer(g1_first, wc1, b1, ln1_g, ln1_b)

    xsa, xsb = _head_kernel(h2, cntp.reshape(-1), set_pad.reshape(-1))
    out = _tc_head(xsa, xsb, merger_W, merger_b.reshape(1, F),
                   ff1_W, ff1_b.reshape(1, F), ff2_W, ff2_b.reshape(1, F))
    return out
